# Initial kernel scaffold; baseline (speedup 1.0000x reference)
import jax
import jax.numpy as jnp
from jax.experimental import pallas as pl


def kernel(target_indices, context_indices, negative_indices, target_weight, context_weight):
    raise NotImplementedError("write your pallas kernel here")


# speedup vs baseline: 1.9094x; 1.9094x over previous
import functools

import jax
import jax.numpy as jnp
from jax import lax
from jax.experimental import pallas as pl
from jax.experimental.pallas import tpu as pltpu
from jax.experimental.pallas import tpu_sc as plsc

B = 16384
V = 1000000
D = 64
C = 20
NNEG = 50
TN = 1 + NNEG
NR = 64

NC = 2
NS = 16
NW = NC * NS
EPW = B // NW
NB = 8
NCH = EPW // NB
LANES = 16
ND = D // LANES

CTX_SLICES = [(0, 80), (80, 80)]
TN_SLICES = [(0, 128), (128, 128), (256, 128), (384, 24)]


def _row(ref, r, dd):
    return ref[r, pl.ds(dd * LANES, LANES)]


def _dot_all_lanes(cv, ref, r, rots):
    p = cv[0] * _row(ref, r, 0)
    for dd in range(1, ND):
        p = p + cv[dd] * _row(ref, r, dd)
    for rot in rots:
        p = p + p.at[rot].get(mode="promise_in_bounds", unique_indices=True)
    return p


def _make_kernel():
    mesh = plsc.VectorSubcoreMesh(core_axis_name="c", subcore_axis_name="s")

    @functools.partial(
        pl.kernel,
        out_type=jax.ShapeDtypeStruct((B, NR), jnp.float32),
        mesh=mesh,
        compiler_params=pltpu.CompilerParams(use_tc_tiling_on_sc=False),
        scratch_types=[
            pltpu.VMEM((NB * C,), jnp.int32),
            pltpu.VMEM((NB * C,), jnp.int32),
            pltpu.VMEM((NB * TN,), jnp.int32),
            pltpu.VMEM((NB * TN,), jnp.int32),
            pltpu.VMEM((NB * C, D), jnp.float32),
            pltpu.VMEM((NB * C, D), jnp.float32),
            pltpu.VMEM((NB * TN, D), jnp.float32),
            pltpu.VMEM((NB * TN, D), jnp.float32),
            pltpu.VMEM((NB, NR), jnp.float32),
            pltpu.SemaphoreType.DMA,
            pltpu.SemaphoreType.DMA,
        ],
    )
    def cbow(ctx_idx_hbm, tn_idx_hbm, tw_hbm, cw_hbm, scores_hbm,
             idx_c0, idx_c1, idx_t0, idx_t1,
             rows_c0, rows_c1, rows_t0, rows_t1,
             scores_buf, semA, semB):
        wid = lax.axis_index("s") * NC + lax.axis_index("c")
        wbase = wid * EPW

        bufs = ((idx_c0, idx_t0, rows_c0, rows_t0, semA),
                (idx_c1, idx_t1, rows_c1, rows_t1, semB))

        def gather_copies(c, buf):
            idx_c, idx_t, rows_c, rows_t, sem = buf
            base = wbase + c * NB
            copies = []
            for (o, l) in CTX_SLICES:
                copies.append(pltpu.make_async_copy(
                    cw_hbm.at[idx_c.at[pl.ds(o, l)]],
                    rows_c.at[pl.ds(o, l)], sem))
            for (o, l) in TN_SLICES:
                copies.append(pltpu.make_async_copy(
                    tw_hbm.at[idx_t.at[pl.ds(o, l)]],
                    rows_t.at[pl.ds(o, l)], sem))
            return base, copies

        def issue(c, buf):
            @pl.when(c < NCH)
            def _():
                idx_c, idx_t, rows_c, rows_t, sem = buf
                base, copies = gather_copies(c, buf)
                pltpu.sync_copy(ctx_idx_hbm.at[pl.ds(base * C, NB * C)], idx_c)
                pltpu.sync_copy(tn_idx_hbm.at[pl.ds(base * TN, NB * TN)], idx_t)
                for cp in copies:
                    cp.start()

        def drain(c, buf):
            _, copies = gather_copies(c, buf)
            for cp in copies:
                cp.wait()

        def compute(c, buf):
            _, _, rows_c, rows_t, _ = buf
            lane = lax.iota(jnp.int32, LANES)
            onehot = [lane == jj for jj in range(LANES)]
            rots = [(lane + sh) & (LANES - 1) for sh in (8, 4, 2, 1)]

            def elem(i, _):
                ri = i * C
                acc = tuple(_row(rows_c, ri, dd) for dd in range(ND))
                for k in range(1, C):
                    acc = tuple(acc[dd] + _row(rows_c, ri + k, dd)
                                for dd in range(ND))
                scale = jnp.float32(1.0 / C)
                cv = tuple(a * scale for a in acc)

                ti = i * TN
                zero = jnp.zeros((LANES,), jnp.float32)
                for g in range(NR // LANES):
                    sv = zero
                    for jj in range(LANES):
                        j = g * LANES + jj
                        if j >= TN:
                            break
                        s = _dot_all_lanes(cv, rows_t, ti + j, rots)
                        sv = jnp.where(onehot[jj], s, sv)
                    scores_buf[i, pl.ds(g * LANES, LANES)] = sv
                return 0

            lax.fori_loop(0, NB, elem, 0)
            gbase = wbase + c * NB
            pltpu.sync_copy(scores_buf, scores_hbm.at[pl.ds(gbase, NB), :])

        issue(jnp.int32(0), bufs[0])
        issue(jnp.int32(1), bufs[1])

        def pair(k, _):
            c0 = 2 * k
            drain(c0, bufs[0])
            compute(c0, bufs[0])
            issue(c0 + 2, bufs[0])
            c1 = c0 + 1
            drain(c1, bufs[1])
            compute(c1, bufs[1])
            issue(c1 + 2, bufs[1])
            return 0

        lax.fori_loop(0, NCH // 2, pair, 0, unroll=False)

    return cbow


_cbow = _make_kernel()


def kernel(target_indices, context_indices, negative_indices, target_weight,
           context_weight):
    ctx_idx = context_indices.astype(jnp.int32).reshape(-1)
    tn_idx = jnp.concatenate(
        [target_indices.astype(jnp.int32),
         negative_indices.astype(jnp.int32)], axis=1).reshape(-1)
    scores = _cbow(ctx_idx, tn_idx, target_weight, context_weight)
    return scores[:, 0:1], scores[:, 1:TN]
